# baseline (speedup 1.0000x reference)
import functools

import jax
import jax.numpy as jnp
from jax import lax
from jax.experimental import pallas as pl
from jax.experimental.pallas import tpu as pltpu
from jax.experimental.pallas import tpu_sc as plsc

_D = 64
_B0 = 4096
_T = 200
_BW = 128
_info = plsc.get_sparse_core_info()
_NC = _info.num_cores
_NS = _info.num_subcores
_NW = _NC * _NS
_NBUF = 5
_NGRP = _T // _NBUF
assert _B0 // _BW == _NW and _T % _NBUF == 0

_mesh = plsc.VectorSubcoreMesh(core_axis_name="c", subcore_axis_name="s")


@functools.partial(
    pl.kernel,
    mesh=_mesh,
    out_type=jax.ShapeDtypeStruct((_T, _D // 8, _B0 // _BW, 8 * _BW),
                                  jnp.float32),
    scratch_types=(
        [pltpu.VMEM((_T, _BW), jnp.int32)]
        + [pltpu.VMEM((_BW, _D), jnp.float32) for _ in range(_NBUF)]
        + [pltpu.VMEM((_D // 8, 8 * _BW), jnp.float32) for _ in range(_NBUF)]
        + [pltpu.SemaphoreType.DMA for _ in range(2 * _NBUF)]
    ),
    compiler_params=pltpu.CompilerParams(use_tc_tiling_on_sc=False,
                                         needs_layout_passes=False),
)
def _gather(table_hbm, idxT_hbm, out_hbm, idx_v, *scratch):
    G = scratch[:_NBUF]
    V = scratch[_NBUF:2 * _NBUF]
    gsems = scratch[2 * _NBUF:3 * _NBUF]
    ssems = scratch[3 * _NBUF:]
    w = lax.axis_index("s") * _NC + lax.axis_index("c")
    pltpu.sync_copy(idxT_hbm.at[:, pl.ds(w * _BW, _BW)], idx_v)
    iota = lax.iota(jnp.int32, 16)

    def fire_gather(t, b):
        pltpu.async_copy(table_hbm.at[idx_v.at[t]], G[b], gsems[b])

    def drain_gather(b):
        pltpu.make_async_copy(table_hbm.at[pl.ds(0, _BW)], G[b],
                              gsems[b]).wait()

    def fire_store(t, b):
        pltpu.async_copy(V[b], out_hbm.at[t, :, w], ssems[b])

    def drain_store(b):
        pltpu.make_async_copy(V[b], out_hbm.at[0, :, w], ssems[b]).wait()

    def transpose(b):
        @plsc.parallel_loop(0, (_BW // 16) * (_D // 16), unroll=4)
        def tbody(tau):
            bl0 = (tau >> 2) * 16
            d0 = (tau & 3) * 16
            blv = bl0 + iota
            for j in range(16):
                rot = (iota + j) & 15
                d = rot + d0
                vec = plsc.load_gather(G[b], [blv, d])
                plsc.store_scatter(V[b], [d >> 3, ((d & 7) << 7) + blv], vec)

    for b in range(_NBUF):
        fire_gather(b, b)

    def body(i, carry):
        for b in range(_NBUF):
            t = i * _NBUF + b
            drain_gather(b)

            @pl.when(i > 0)
            def _ds():
                drain_store(b)

            transpose(b)

            @pl.when(i < _NGRP - 1)
            def _fg():
                fire_gather(t + _NBUF, b)

            fire_store(t, b)
        return carry

    lax.fori_loop(0, _NGRP, body, 0)
    for b in range(_NBUF):
        drain_store(b)


def kernel(edge_type, table):
    idxT = edge_type.astype(jnp.int32).T
    out4 = _gather(table, idxT)
    out5 = out4.reshape(_T, _D // 8, _B0 // _BW, 8, _BW)
    return out5.transpose(2, 4, 0, 1, 3).reshape(_B0, _T, _D)


# speedup vs baseline: 1.5963x; 1.5963x over previous
import functools

import jax
import jax.numpy as jnp
from jax import lax
from jax.experimental import pallas as pl
from jax.experimental.pallas import tpu as pltpu
from jax.experimental.pallas import tpu_sc as plsc

_D = 64
_B0 = 4096
_T = 200
_BW = 128
_info = plsc.get_sparse_core_info()
_NC = _info.num_cores
_NS = _info.num_subcores
_NW = _NC * _NS
_NBUF = 4
_NGRP = _T // _NBUF
assert _B0 // _BW == _NW and _T % _NBUF == 0

_mesh = plsc.VectorSubcoreMesh(core_axis_name="c", subcore_axis_name="s")


@functools.partial(
    pl.kernel,
    mesh=_mesh,
    out_type=jax.ShapeDtypeStruct((_T, _D // 8, _B0 // _BW, 8 * _BW),
                                  jnp.float32),
    scratch_types=(
        [pltpu.VMEM((_T, _BW), jnp.int32)]
        + [pltpu.VMEM((_BW, _D), jnp.float32) for _ in range(_NBUF)]
        + [pltpu.VMEM((_D // 8, 8 * _BW), jnp.float32) for _ in range(_NBUF)]
        + [pltpu.SemaphoreType.DMA for _ in range(2 * _NBUF)]
    ),
    compiler_params=pltpu.CompilerParams(use_tc_tiling_on_sc=False,
                                         needs_layout_passes=False),
)
def _gather(table_hbm, idxT_hbm, out_hbm, idx_v, *scratch):
    G = scratch[:_NBUF]
    V = scratch[_NBUF:2 * _NBUF]
    gsems = scratch[2 * _NBUF:3 * _NBUF]
    ssems = scratch[3 * _NBUF:]
    w = lax.axis_index("s") * _NC + lax.axis_index("c")
    pltpu.sync_copy(idxT_hbm.at[:, pl.ds(w * _BW, _BW)], idx_v)
    iota = lax.iota(jnp.int32, 16)

    def fire_gather(t, b):
        pltpu.async_copy(table_hbm.at[idx_v.at[t]], G[b], gsems[b])

    def drain_gather(b):
        pltpu.make_async_copy(table_hbm.at[pl.ds(0, _BW)], G[b],
                              gsems[b]).wait()

    def fire_store(t, b):
        pltpu.async_copy(V[b], out_hbm.at[t, :, w], ssems[b])

    def drain_store(b):
        pltpu.make_async_copy(V[b], out_hbm.at[0, :, w], ssems[b]).wait()

    def transpose(b):
        @plsc.parallel_loop(0, (_BW // 16) * (_D // 16), unroll=2)
        def tbody(tau):
            bl0 = (tau >> 2) * 16
            d0 = (tau & 3) * 16
            blv = bl0 + iota
            for j in range(16):
                rot = (iota + j) & 15
                d = rot + d0
                vec = plsc.load_gather(G[b], [blv, d])
                plsc.store_scatter(V[b], [d >> 3, ((d & 7) << 7) + blv], vec)

    for b in range(_NBUF):
        fire_gather(b, b)

    def body(i, carry):
        for b in range(_NBUF):
            t = i * _NBUF + b
            drain_gather(b)

            @pl.when(i > 0)
            def _ds():
                drain_store(b)

            transpose(b)

            @pl.when(i < _NGRP - 1)
            def _fg():
                fire_gather(t + _NBUF, b)

            fire_store(t, b)
        return carry

    lax.fori_loop(0, _NGRP, body, 0)
    for b in range(_NBUF):
        drain_store(b)


def kernel(edge_type, table):
    idxT = edge_type.astype(jnp.int32).T
    out4 = _gather(table, idxT)
    out5 = out4.reshape(_T, _D // 8, _B0 // _BW, 8, _BW)
    return out5.transpose(2, 4, 0, 1, 3).reshape(_B0, _T, _D)
